# baseline (speedup 1.0000x reference)
import functools

import jax
import jax.numpy as jnp
from jax import lax
from jax.experimental import pallas as pl
from jax.experimental.pallas import tpu as pltpu
from jax.experimental.pallas import tpu_sc as plsc

NC = 2
NS = 16
NW = NC * NS
CH = 128
KG = 5
SC_ROWS = CH * KG

VCHUNK = 1024
BCHUNK = 512


def _tc_table_transpose(d, v):
    grid = (pl.cdiv(v, VCHUNK),)

    def body(src, dst):
        dst[...] = jnp.transpose(src[...], (1, 0))

    return pl.pallas_call(
        body,
        grid=grid,
        in_specs=[pl.BlockSpec((d, VCHUNK), lambda i: (0, i))],
        out_specs=pl.BlockSpec((VCHUNK, d), lambda i: (i, 0)),
        out_shape=jax.ShapeDtypeStruct((v, d), jnp.float32),
    )


def _tc_out_relayout(b, h, d):
    grid = (b // BCHUNK,)

    def body(src, dst):
        for hh in range(h):
            dst[hh, :, :] = jnp.transpose(src[:, hh, :], (1, 0))

    return pl.pallas_call(
        body,
        grid=grid,
        in_specs=[pl.BlockSpec((BCHUNK, h, d), lambda j: (j, 0, 0))],
        out_specs=pl.BlockSpec((h, d, BCHUNK), lambda j: (0, 0, j)),
        out_shape=jax.ShapeDtypeStruct((h, d, b), jnp.float32),
    )


def _sc_gather(n_super, d):
    mesh = plsc.VectorSubcoreMesh(core_axis_name="c", subcore_axis_name="s")

    @functools.partial(
        pl.kernel,
        mesh=mesh,
        compiler_params=pltpu.CompilerParams(use_tc_tiling_on_sc=False),
        out_type=jax.ShapeDtypeStruct((NW, n_super, SC_ROWS, d), jnp.float32),
        scratch_types=[
            pltpu.VMEM((n_super * KG, CH), jnp.int32),
            pltpu.VMEM((2, SC_ROWS, d), jnp.float32),
            pltpu.SemaphoreType.DMA,
            pltpu.SemaphoreType.DMA,
        ],
    )
    def k(x_hbm, table_hbm, out_hbm, idx_v, rows_v, sem0, sem1):
        wid = lax.axis_index("s") * NC + lax.axis_index("c")
        pltpu.sync_copy(x_hbm.at[wid], idx_v)
        sems = (sem0, sem1)

        def fire(s, b):
            for j in range(KG):
                pltpu.async_copy(
                    table_hbm.at[idx_v.at[s * KG + j]],
                    rows_v.at[b, pl.ds(j * CH, CH)],
                    sems[b],
                )

        def drain_store(s, b):
            for j in range(KG):
                pltpu.make_async_copy(
                    table_hbm.at[idx_v.at[s * KG + j]],
                    rows_v.at[b, pl.ds(j * CH, CH)],
                    sems[b],
                ).wait()
            pltpu.sync_copy(rows_v.at[b], out_hbm.at[wid, s])

        fire(0, 0)
        fire(1, 1)

        def body(i, carry):
            s0 = 2 * i
            drain_store(s0, 0)
            fire(s0 + 2, 0)
            drain_store(s0 + 1, 1)
            fire(s0 + 3, 1)
            return carry

        lax.fori_loop(0, n_super // 2 - 1, body, 0)
        drain_store(n_super - 2, 0)
        drain_store(n_super - 1, 1)

    return k


def kernel(x, table):
    b, h = x.shape
    v, d = table.shape
    n = b * h
    assert n % (NW * SC_ROWS) == 0
    n_super = n // (NW * SC_ROWS)
    assert n_super % 2 == 0 and b % BCHUNK == 0

    table_rm = _tc_table_transpose(d, v)(jnp.transpose(table))
    xr = jnp.asarray(x, jnp.int32).reshape(NW, n_super * KG, CH)
    g = _sc_gather(n_super, d)(xr, table_rm)
    outt = _tc_out_relayout(b, h, d)(g.reshape(b, h, d))
    return jnp.transpose(outt, (2, 0, 1))


# speedup vs baseline: 1.2518x; 1.2518x over previous
import functools

import jax
import jax.numpy as jnp
from jax import lax
from jax.experimental import pallas as pl
from jax.experimental.pallas import tpu as pltpu
from jax.experimental.pallas import tpu_sc as plsc

NC = 2
NS = 16
NW = NC * NS
BW = 128
L = 16


def _transpose_block(gbuf, tbuf, d):
    rows = lax.iota(jnp.int32, L)
    for f in range(d):
        fv = jnp.full((L,), f, jnp.int32)
        for m in range(BW // L):
            v = plsc.load_gather(gbuf, [rows + (m * L), fv])
            tbuf[f, pl.ds(m * L, L)] = v


def _emb_kernel(b, h, d):
    mesh = plsc.VectorSubcoreMesh(core_axis_name="c", subcore_axis_name="s")

    @functools.partial(
        pl.kernel,
        mesh=mesh,
        compiler_params=pltpu.CompilerParams(
            use_tc_tiling_on_sc=False, needs_layout_passes=False
        ),
        out_type=jax.ShapeDtypeStruct((h, d, b), jnp.float32),
        scratch_types=[
            pltpu.VMEM((h, BW), jnp.int32),
            pltpu.VMEM((2, BW, d), jnp.float32),
            pltpu.VMEM((2, d, BW), jnp.float32),
            pltpu.SemaphoreType.DMA,
            pltpu.SemaphoreType.DMA,
        ],
    )
    def k(xt_hbm, table_hbm, out_hbm, idx_v, gbuf, tbuf, sem0, sem1):
        wid = lax.axis_index("s") * NC + lax.axis_index("c")
        b0 = wid * BW
        pltpu.sync_copy(xt_hbm.at[:, pl.ds(b0, BW)], idx_v)
        sems = (sem0, sem1)

        def fire(hh, buf):
            pltpu.async_copy(table_hbm.at[idx_v.at[hh]], gbuf.at[buf], sems[buf])

        def drain(hh, buf):
            pltpu.make_async_copy(
                table_hbm.at[idx_v.at[hh]], gbuf.at[buf], sems[buf]
            ).wait()

        def emit(hh, buf):
            drain(hh, buf)
            _transpose_block(gbuf.at[buf], tbuf.at[buf], d)
            pltpu.sync_copy(tbuf.at[buf], out_hbm.at[hh, :, pl.ds(b0, BW)])

        fire(0, 0)
        fire(1, 1)

        def body(i, carry):
            h2 = 2 * i
            emit(h2, 0)
            fire(h2 + 2, 0)
            emit(h2 + 1, 1)
            fire(h2 + 3, 1)
            return carry

        lax.fori_loop(0, h // 2 - 1, body, 0)
        emit(h - 2, 0)
        emit(h - 1, 1)

    return k


def kernel(x, table):
    b, h = x.shape
    v, d = table.shape
    assert b % NW == 0 and b // NW == BW and h % 2 == 0
    xt = jnp.transpose(jnp.asarray(x, jnp.int32))
    outt = _emb_kernel(b, h, d)(xt, table)
    return jnp.transpose(outt, (2, 0, 1))


# speedup vs baseline: 1.5529x; 1.2405x over previous
import functools


import jax
import jax.numpy as jnp
from jax import lax
from jax.experimental import pallas as pl
from jax.experimental.pallas import tpu as pltpu
from jax.experimental.pallas import tpu_sc as plsc

NC = 2
NS = 16
NW = NC * NS
BW = 128
L = 16


def _transpose_block(gbuf, tbuf, d):
    base = lax.iota(jnp.int32, L)
    diags = [(base + s) % L for s in range(L)]

    def body(t, carry):
        jv = base + t * L
        for f0 in range(0, d, L):
            for s in range(L):
                fv = diags[s] + f0
                v = plsc.load_gather(gbuf, [jv, fv])
                plsc.store_scatter(tbuf, [fv, jv], v)
        return carry

    lax.fori_loop(0, BW // L, body, 0)


def _emb_kernel(b, h, d):
    mesh = plsc.VectorSubcoreMesh(core_axis_name="c", subcore_axis_name="s")

    @functools.partial(
        pl.kernel,
        mesh=mesh,
        compiler_params=pltpu.CompilerParams(
            use_tc_tiling_on_sc=False, needs_layout_passes=False
        ),
        out_type=jax.ShapeDtypeStruct((h, d, b), jnp.float32),
        scratch_types=[
            pltpu.VMEM((h, BW), jnp.int32),
            pltpu.VMEM((2, BW, d), jnp.float32),
            pltpu.VMEM((2, d, BW), jnp.float32),
            pltpu.SemaphoreType.DMA,
            pltpu.SemaphoreType.DMA,
        ],
    )
    def k(xt_hbm, table_hbm, out_hbm, idx_v, gbuf, tbuf, sem0, sem1):
        wid = lax.axis_index("s") * NC + lax.axis_index("c")
        b0 = wid * BW
        pltpu.sync_copy(xt_hbm.at[:, pl.ds(b0, BW)], idx_v)
        sems = (sem0, sem1)

        def fire(hh, buf):
            pltpu.async_copy(table_hbm.at[idx_v.at[hh]], gbuf.at[buf], sems[buf])

        def drain(hh, buf):
            pltpu.make_async_copy(
                table_hbm.at[idx_v.at[hh]], gbuf.at[buf], sems[buf]
            ).wait()

        def emit(hh, buf):
            drain(hh, buf)
            _transpose_block(gbuf.at[buf], tbuf.at[buf], d)
            pltpu.sync_copy(tbuf.at[buf], out_hbm.at[hh, :, pl.ds(b0, BW)])

        fire(0, 0)
        fire(1, 1)

        def body(i, carry):
            h2 = 2 * i
            emit(h2, 0)
            fire(h2 + 2, 0)
            emit(h2 + 1, 1)
            fire(h2 + 3, 1)
            return carry

        lax.fori_loop(0, h // 2 - 1, body, 0)
        emit(h - 2, 0)
        emit(h - 1, 1)

    return k


def kernel(x, table):
    b, h = x.shape
    v, d = table.shape
    assert b % NW == 0 and b // NW == BW and h % 2 == 0
    xt = jnp.transpose(jnp.asarray(x, jnp.int32))
    outt = _emb_kernel(b, h, d)(xt, table)
    return jnp.transpose(outt, (2, 0, 1))


# speedup vs baseline: 1.7565x; 1.1311x over previous
import functools


import jax
import jax.numpy as jnp
from jax import lax
from jax.experimental import pallas as pl
from jax.experimental.pallas import tpu as pltpu
from jax.experimental.pallas import tpu_sc as plsc

NC = 2
NS = 16
NW = NC * NS
BW = 128
L = 16


def _transpose_block(gbuf, tbuf, d):
    base = lax.iota(jnp.int32, L)
    diags = [(base + s) % L for s in range(L)]

    def body(t, carry):
        jv = base + t * L
        for f0 in range(0, d, L):
            fvs = [diags[s] + f0 for s in range(L)]
            vs = [plsc.load_gather(gbuf, [jv, fv]) for fv in fvs]
            for fv, v in zip(fvs, vs):
                plsc.store_scatter(tbuf, [fv, jv], v)
        return carry

    lax.fori_loop(0, BW // L, body, 0)


def _emb_kernel(b, h, d):
    mesh = plsc.VectorSubcoreMesh(core_axis_name="c", subcore_axis_name="s")

    @functools.partial(
        pl.kernel,
        mesh=mesh,
        compiler_params=pltpu.CompilerParams(
            use_tc_tiling_on_sc=False, needs_layout_passes=False
        ),
        out_type=jax.ShapeDtypeStruct((h, d, b), jnp.float32),
        scratch_types=[
            pltpu.VMEM((h, BW), jnp.int32),
            pltpu.VMEM((2, BW, d), jnp.float32),
            pltpu.VMEM((2, d, BW), jnp.float32),
            pltpu.SemaphoreType.DMA,
            pltpu.SemaphoreType.DMA,
        ],
    )
    def k(xt_hbm, table_hbm, out_hbm, idx_v, gbuf, tbuf, sem0, sem1):
        wid = lax.axis_index("s") * NC + lax.axis_index("c")
        b0 = wid * BW
        pltpu.sync_copy(xt_hbm.at[:, pl.ds(b0, BW)], idx_v)
        sems = (sem0, sem1)

        def fire(hh, buf):
            pltpu.async_copy(table_hbm.at[idx_v.at[hh]], gbuf.at[buf], sems[buf])

        def drain(hh, buf):
            pltpu.make_async_copy(
                table_hbm.at[idx_v.at[hh]], gbuf.at[buf], sems[buf]
            ).wait()

        def emit(hh, buf):
            drain(hh, buf)
            _transpose_block(gbuf.at[buf], tbuf.at[buf], d)
            pltpu.sync_copy(tbuf.at[buf], out_hbm.at[hh, :, pl.ds(b0, BW)])

        fire(0, 0)
        fire(1, 1)

        def body(i, carry):
            h2 = 2 * i
            emit(h2, 0)
            fire(h2 + 2, 0)
            emit(h2 + 1, 1)
            fire(h2 + 3, 1)
            return carry

        lax.fori_loop(0, h // 2 - 1, body, 0)
        emit(h - 2, 0)
        emit(h - 1, 1)

    return k


def kernel(x, table):
    b, h = x.shape
    v, d = table.shape
    assert b % NW == 0 and b // NW == BW and h % 2 == 0
    xt = jnp.transpose(jnp.asarray(x, jnp.int32))
    outt = _emb_kernel(b, h, d)(xt, table)
    return jnp.transpose(outt, (2, 0, 1))


# speedup vs baseline: 1.8706x; 1.0649x over previous
import functools

import jax
import jax.numpy as jnp
from jax import lax
from jax.experimental import pallas as pl
from jax.experimental.pallas import tpu as pltpu
from jax.experimental.pallas import tpu_sc as plsc

NC = 2
NS = 16
NW = NC * NS
BW = 128
L = 16


def _transpose_block(gbuf, off, tbuf, d):
    base = lax.iota(jnp.int32, L)
    diags = [(base + s) % L for s in range(L)]

    def body(t, carry):
        jv = base + t * L
        ov = plsc.load_gather(off, [jv])
        for f0 in range(0, d, L):
            fvs = [diags[s] + f0 for s in range(L)]
            vs = [plsc.load_gather(gbuf, [jv, fv + ov]) for fv in fvs]
            for fv, v in zip(fvs, vs):
                plsc.store_scatter(tbuf, [fv, jv], v)
        return carry

    lax.fori_loop(0, BW // L, body, 0)


def _emb_kernel(b, h, d):
    mesh = plsc.VectorSubcoreMesh(core_axis_name="c", subcore_axis_name="s")

    @functools.partial(
        pl.kernel,
        mesh=mesh,
        compiler_params=pltpu.CompilerParams(
            use_tc_tiling_on_sc=True, needs_layout_passes=False
        ),
        out_type=jax.ShapeDtypeStruct((h, d, b), jnp.float32),
        scratch_types=[
            pltpu.VMEM((h, BW), jnp.int32),
            pltpu.VMEM((h, BW), jnp.int32),
            pltpu.VMEM((h, BW), jnp.int32),
            pltpu.VMEM((2, BW, 2 * d), jnp.float32),
            pltpu.VMEM((2, d, BW), jnp.float32),
            pltpu.SemaphoreType.DMA,
            pltpu.SemaphoreType.DMA,
        ],
    )
    def k(xt_hbm, tp_hbm, out_hbm, idx_v, pidx_v, off_v, gbuf, tbuf, sem0, sem1):
        wid = lax.axis_index("s") * NC + lax.axis_index("c")
        b0 = wid * BW
        pltpu.sync_copy(xt_hbm.at[:, pl.ds(b0, BW)], idx_v)

        def split(hh, carry):
            for m in range(BW // L):
                v = idx_v[hh, pl.ds(m * L, L)]
                pidx_v[hh, pl.ds(m * L, L)] = v >> 1
                off_v[hh, pl.ds(m * L, L)] = (v & 1) * d
            return carry

        lax.fori_loop(0, h, split, 0)
        sems = (sem0, sem1)

        def fire(hh, buf):
            pltpu.async_copy(tp_hbm.at[pidx_v.at[hh]], gbuf.at[buf], sems[buf])

        def drain(hh, buf):
            pltpu.make_async_copy(
                tp_hbm.at[pidx_v.at[hh]], gbuf.at[buf], sems[buf]
            ).wait()

        def emit(hh, buf):
            drain(hh, buf)
            _transpose_block(gbuf.at[buf], off_v.at[hh], tbuf.at[buf], d)
            pltpu.sync_copy(tbuf.at[buf], out_hbm.at[hh, :, pl.ds(b0, BW)])

        fire(0, 0)
        fire(1, 1)

        def body(i, carry):
            h2 = 2 * i
            emit(h2, 0)
            fire(h2 + 2, 0)
            emit(h2 + 1, 1)
            fire(h2 + 3, 1)
            return carry

        lax.fori_loop(0, h // 2 - 1, body, 0)
        emit(h - 2, 0)
        emit(h - 1, 1)

    return k


def kernel(x, table):
    b, h = x.shape
    v, d = table.shape
    assert b % NW == 0 and b // NW == BW and h % 2 == 0 and v % 2 == 0
    xt = jnp.transpose(jnp.asarray(x, jnp.int32))
    table_pairs = table.reshape(v // 2, 2 * d)
    outt = _emb_kernel(b, h, d)(xt, table_pairs)
    return jnp.transpose(outt, (2, 0, 1))


# speedup vs baseline: 3.1977x; 1.7095x over previous
import functools

import jax
import jax.numpy as jnp
from jax import lax
from jax.experimental import pallas as pl
from jax.experimental.pallas import tpu as pltpu
from jax.experimental.pallas import tpu_sc as plsc

NC = 2
NS = 16
NW = NC * NS
BW = 128
L = 16

TV = 384
TP = TV // 2
TS = 512


def _relayout_kernel(d, v):
    n_full = v // TV
    rem_v = v - n_full * TV
    per_w = n_full // NW
    extra = n_full - per_w * NW
    mesh = plsc.VectorSubcoreMesh(core_axis_name="c", subcore_axis_name="s")

    @functools.partial(
        pl.kernel,
        mesh=mesh,
        compiler_params=pltpu.CompilerParams(
            use_tc_tiling_on_sc=True, needs_layout_passes=False
        ),
        out_type=jax.ShapeDtypeStruct((v // 2, 2 * d), jnp.float32),
        scratch_types=[
            pltpu.VMEM((2, d, TS), jnp.float32),
            pltpu.VMEM((2, TP, 2 * d), jnp.float32),
            pltpu.SemaphoreType.DMA,
            pltpu.SemaphoreType.DMA,
        ],
    )
    def k(tt_hbm, tail_hbm, out_hbm, slab, tbuf, sem0, sem1):
        wid = lax.axis_index("s") * NC + lax.axis_index("c")
        nb = per_w + jnp.where(wid < extra, 1, 0)
        sems = (sem0, sem1)
        base = lax.iota(jnp.int32, L)
        diags = [(base + s) % L for s in range(L)]

        def blk(t):
            return wid + t * NW

        def fire(t, buf):
            c0 = blk(t) * TV
            pltpu.async_copy(
                tt_hbm.at[:, pl.ds(c0, TV)],
                slab.at[buf, :, pl.ds(0, TV)],
                sems[buf],
            )

        def drain(t, buf):
            c0 = blk(t) * TV
            pltpu.make_async_copy(
                tt_hbm.at[:, pl.ds(c0, TV)],
                slab.at[buf, :, pl.ds(0, TV)],
                sems[buf],
            ).wait()

        def transpose(buf):
            sl = slab.at[buf]
            tb = tbuf.at[buf]

            def body(p0, carry):
                pv = base + p0 * L
                cv0 = pv * 2
                for j in (0, 1):
                    cv = cv0 + j
                    for f0 in range(0, d, L):
                        fvs = [diags[s] + f0 for s in range(L)]
                        vs = [plsc.load_gather(sl, [fv, cv]) for fv in fvs]
                        for fv, vv in zip(fvs, vs):
                            plsc.store_scatter(tb, [pv, fv + (j * d)], vv)
                return carry

            lax.fori_loop(0, TP // L, body, 0)

        def emit(t, buf):
            drain(t, buf)
            transpose(buf)
            pltpu.sync_copy(tbuf.at[buf], out_hbm.at[pl.ds(blk(t) * TP, TP)])

        fire(0, 0)
        fire(1, 1)

        def body(i, carry):
            t0 = 2 * i
            emit(t0, 0)

            @pl.when(t0 + 2 < nb)
            def _():
                fire(t0 + 2, 0)

            emit(t0 + 1, 1)

            @pl.when(t0 + 3 < nb)
            def _():
                fire(t0 + 3, 1)

            return carry

        lax.fori_loop(0, nb // 2, body, 0)

        @pl.when(nb % 2 == 1)
        def _():
            emit(nb - 1, 0)

        if rem_v:
            @pl.when(wid == NW - 1)
            def _():
                pltpu.sync_copy(
                    tail_hbm, out_hbm.at[pl.ds(n_full * TP, rem_v // 2)]
                )

    return k


def _gather_kernel(b, h, d):
    mesh = plsc.VectorSubcoreMesh(core_axis_name="c", subcore_axis_name="s")

    @functools.partial(
        pl.kernel,
        mesh=mesh,
        compiler_params=pltpu.CompilerParams(
            use_tc_tiling_on_sc=True, needs_layout_passes=False
        ),
        out_type=jax.ShapeDtypeStruct((h, d, b), jnp.float32),
        scratch_types=[
            pltpu.VMEM((h, BW), jnp.int32),
            pltpu.VMEM((h, BW), jnp.int32),
            pltpu.VMEM((h, BW), jnp.int32),
            pltpu.VMEM((2, BW, 2 * d), jnp.float32),
            pltpu.VMEM((2, d, BW), jnp.float32),
            pltpu.SemaphoreType.DMA,
            pltpu.SemaphoreType.DMA,
        ],
    )
    def k(xt_hbm, tp_hbm, out_hbm, idx_v, pidx_v, off_v, gbuf, tbuf, sem0, sem1):
        wid = lax.axis_index("s") * NC + lax.axis_index("c")
        b0 = wid * BW
        pltpu.sync_copy(xt_hbm.at[:, pl.ds(b0, BW)], idx_v)

        def split(hh, carry):
            for m in range(BW // L):
                v = idx_v[hh, pl.ds(m * L, L)]
                pidx_v[hh, pl.ds(m * L, L)] = v >> 1
                off_v[hh, pl.ds(m * L, L)] = (v & 1) * d
            return carry

        lax.fori_loop(0, h, split, 0)
        sems = (sem0, sem1)
        base = lax.iota(jnp.int32, L)
        diags = [(base + s) % L for s in range(L)]

        def fire(hh, buf):
            pltpu.async_copy(tp_hbm.at[pidx_v.at[hh]], gbuf.at[buf], sems[buf])

        def drain(hh, buf):
            pltpu.make_async_copy(
                tp_hbm.at[pidx_v.at[hh]], gbuf.at[buf], sems[buf]
            ).wait()

        def transpose(hh, buf):
            gb = gbuf.at[buf]
            tb = tbuf.at[buf]
            off = off_v.at[hh]

            def body(t, carry):
                jv = base + t * L
                ov = plsc.load_gather(off, [jv])
                for f0 in range(0, d, L):
                    fvs = [diags[s] + f0 for s in range(L)]
                    vs = [plsc.load_gather(gb, [jv, fv + ov]) for fv in fvs]
                    for fv, vv in zip(fvs, vs):
                        plsc.store_scatter(tb, [fv, jv], vv)
                return carry

            lax.fori_loop(0, BW // L, body, 0)

        def emit(hh, buf):
            drain(hh, buf)
            transpose(hh, buf)
            pltpu.sync_copy(tbuf.at[buf], out_hbm.at[hh, :, pl.ds(b0, BW)])

        fire(0, 0)
        fire(1, 1)

        def body(i, carry):
            h2 = 2 * i
            emit(h2, 0)
            fire(h2 + 2, 0)
            emit(h2 + 1, 1)
            fire(h2 + 3, 1)
            return carry

        lax.fori_loop(0, h // 2 - 1, body, 0)
        emit(h - 2, 0)
        emit(h - 1, 1)

    return k


def kernel(x, table):
    b, h = x.shape
    v, d = table.shape
    assert b % NW == 0 and b // NW == BW and h % 2 == 0 and v % 2 == 0
    xt = jnp.transpose(jnp.asarray(x, jnp.int32))
    tt = jnp.transpose(table)
    n_full = (v // TV) * TV
    tail_pairs = table[n_full:].reshape((v - n_full) // 2, 2 * d)
    table_pairs = _relayout_kernel(d, v)(tt, tail_pairs)
    outt = _gather_kernel(b, h, d)(xt, table_pairs)
    return jnp.transpose(outt, (2, 0, 1))


# speedup vs baseline: 3.2107x; 1.0041x over previous
import functools

import jax
import jax.numpy as jnp
from jax import lax
from jax.experimental import pallas as pl
from jax.experimental.pallas import tpu as pltpu
from jax.experimental.pallas import tpu_sc as plsc

NC = 2
NS = 16
NW = NC * NS
BW = 128
L = 16

TV = 384
TP = TV // 2
TS = 389


def _relayout_kernel(d, v):
    n_full = v // TV
    rem_v = v - n_full * TV
    per_w = n_full // NW
    extra = n_full - per_w * NW
    mesh = plsc.VectorSubcoreMesh(core_axis_name="c", subcore_axis_name="s")

    @functools.partial(
        pl.kernel,
        mesh=mesh,
        compiler_params=pltpu.CompilerParams(
            use_tc_tiling_on_sc=True, needs_layout_passes=False
        ),
        out_type=jax.ShapeDtypeStruct((v // 2, 2 * d), jnp.float32),
        scratch_types=[
            pltpu.VMEM((2, d, TS), jnp.float32),
            pltpu.VMEM((2, TP, 2 * d), jnp.float32),
            pltpu.SemaphoreType.DMA,
            pltpu.SemaphoreType.DMA,
        ],
    )
    def k(tt_hbm, tail_hbm, out_hbm, slab, tbuf, sem0, sem1):
        wid = lax.axis_index("s") * NC + lax.axis_index("c")
        nb = per_w + jnp.where(wid < extra, 1, 0)
        sems = (sem0, sem1)
        base = lax.iota(jnp.int32, L)
        diags = [(base + s) % L for s in range(L)]

        def blk(t):
            return wid + t * NW

        def fire(t, buf):
            c0 = blk(t) * TV
            pltpu.async_copy(
                tt_hbm.at[:, pl.ds(c0, TV)],
                slab.at[buf, :, pl.ds(0, TV)],
                sems[buf],
            )

        def drain(t, buf):
            c0 = blk(t) * TV
            pltpu.make_async_copy(
                tt_hbm.at[:, pl.ds(c0, TV)],
                slab.at[buf, :, pl.ds(0, TV)],
                sems[buf],
            ).wait()

        def transpose(buf):
            sl = slab.at[buf]
            tb = tbuf.at[buf]

            def body(p0, carry):
                pv = base + p0 * L
                cv0 = pv * 2
                for j in (0, 1):
                    cv = cv0 + j
                    for f0 in range(0, d, L):
                        fvs = [diags[s] + f0 for s in range(L)]
                        vs = [plsc.load_gather(sl, [fv, cv]) for fv in fvs]
                        for fv, vv in zip(fvs, vs):
                            plsc.store_scatter(tb, [pv, fv + (j * d)], vv)
                return carry

            lax.fori_loop(0, TP // L, body, 0)

        def emit(t, buf):
            drain(t, buf)
            transpose(buf)
            pltpu.sync_copy(tbuf.at[buf], out_hbm.at[pl.ds(blk(t) * TP, TP)])

        fire(0, 0)
        fire(1, 1)

        def body(i, carry):
            t0 = 2 * i
            emit(t0, 0)

            @pl.when(t0 + 2 < nb)
            def _():
                fire(t0 + 2, 0)

            emit(t0 + 1, 1)

            @pl.when(t0 + 3 < nb)
            def _():
                fire(t0 + 3, 1)

            return carry

        lax.fori_loop(0, nb // 2, body, 0)

        @pl.when(nb % 2 == 1)
        def _():
            emit(nb - 1, 0)

        if rem_v:
            @pl.when(wid == NW - 1)
            def _():
                pltpu.sync_copy(
                    tail_hbm, out_hbm.at[pl.ds(n_full * TP, rem_v // 2)]
                )

    return k


def _gather_kernel(b, h, d):
    mesh = plsc.VectorSubcoreMesh(core_axis_name="c", subcore_axis_name="s")

    @functools.partial(
        pl.kernel,
        mesh=mesh,
        compiler_params=pltpu.CompilerParams(
            use_tc_tiling_on_sc=True, needs_layout_passes=False
        ),
        out_type=jax.ShapeDtypeStruct((h, d, b), jnp.float32),
        scratch_types=[
            pltpu.VMEM((h, BW), jnp.int32),
            pltpu.VMEM((h, BW), jnp.int32),
            pltpu.VMEM((h, BW), jnp.int32),
            pltpu.VMEM((2, BW, 2 * d), jnp.float32),
            pltpu.VMEM((2, d, BW), jnp.float32),
            pltpu.SemaphoreType.DMA,
            pltpu.SemaphoreType.DMA,
        ],
    )
    def k(xt_hbm, tp_hbm, out_hbm, idx_v, pidx_v, off_v, gbuf, tbuf, sem0, sem1):
        wid = lax.axis_index("s") * NC + lax.axis_index("c")
        b0 = wid * BW
        pltpu.sync_copy(xt_hbm.at[:, pl.ds(b0, BW)], idx_v)

        def split(hh, carry):
            for m in range(BW // L):
                v = idx_v[hh, pl.ds(m * L, L)]
                pidx_v[hh, pl.ds(m * L, L)] = v >> 1
                off_v[hh, pl.ds(m * L, L)] = (v & 1) * d
            return carry

        lax.fori_loop(0, h, split, 0)
        sems = (sem0, sem1)
        base = lax.iota(jnp.int32, L)
        diags = [(base + s) % L for s in range(L)]

        def fire(hh, buf):
            pltpu.async_copy(tp_hbm.at[pidx_v.at[hh]], gbuf.at[buf], sems[buf])

        def drain(hh, buf):
            pltpu.make_async_copy(
                tp_hbm.at[pidx_v.at[hh]], gbuf.at[buf], sems[buf]
            ).wait()

        def transpose(hh, buf):
            gb = gbuf.at[buf]
            tb = tbuf.at[buf]
            off = off_v.at[hh]

            def body(t, carry):
                jv = base + t * L
                ov = plsc.load_gather(off, [jv])
                for f0 in range(0, d, L):
                    fvs = [diags[s] + f0 for s in range(L)]
                    vs = [plsc.load_gather(gb, [jv, fv + ov]) for fv in fvs]
                    for fv, vv in zip(fvs, vs):
                        plsc.store_scatter(tb, [fv, jv], vv)
                return carry

            lax.fori_loop(0, BW // L, body, 0)

        def emit(hh, buf):
            drain(hh, buf)
            transpose(hh, buf)
            pltpu.sync_copy(tbuf.at[buf], out_hbm.at[hh, :, pl.ds(b0, BW)])

        fire(0, 0)
        fire(1, 1)

        def body(i, carry):
            h2 = 2 * i
            emit(h2, 0)
            fire(h2 + 2, 0)
            emit(h2 + 1, 1)
            fire(h2 + 3, 1)
            return carry

        lax.fori_loop(0, h // 2 - 1, body, 0)
        emit(h - 2, 0)
        emit(h - 1, 1)

    return k


def kernel(x, table):
    b, h = x.shape
    v, d = table.shape
    assert b % NW == 0 and b // NW == BW and h % 2 == 0 and v % 2 == 0
    xt = jnp.transpose(jnp.asarray(x, jnp.int32))
    tt = jnp.transpose(table)
    n_full = (v // TV) * TV
    tail_pairs = table[n_full:].reshape((v - n_full) // 2, 2 * d)
    table_pairs = _relayout_kernel(d, v)(tt, tail_pairs)
    outt = _gather_kernel(b, h, d)(xt, table_pairs)
    return jnp.transpose(outt, (2, 0, 1))
